# Initial kernel scaffold; baseline (speedup 1.0000x reference)
import functools

import jax
import jax.numpy as jnp
from jax import lax
from jax.experimental import pallas as pl
from jax.experimental.pallas import tpu as pltpu
from jax.experimental.pallas import tpu_sc as plsc

B, C, NTOT = 1024, 16, 4096
NVEC = NTOT // 16
PASS_SHIFTS = (0, 11, 22)
PASS_BITS = (11, 11, 10)
NBINS_MAX = 1 << 11


def _digit(u, shift, nbits):
    mask = (1 << nbits) - 1
    return jnp.bitwise_and(lax.shift_right_logical(u, jnp.int32(shift)),
                           jnp.int32(mask))


def _sc_sort_gather(d_f2):
    info = plsc.get_sparse_core_info()
    nc, ns = info.num_cores, info.num_subcores
    nw = nc * ns
    assert B % nw == 0
    rows_per_w = B // nw

    mesh = plsc.VectorSubcoreMesh(core_axis_name="c", subcore_axis_name="s")

    @functools.partial(
        pl.kernel,
        mesh=mesh,
        out_type=jax.ShapeDtypeStruct((B * C, NTOT), jnp.float32),
        scratch_types=[
            pltpu.VMEM((NTOT,), jnp.float32),
            pltpu.VMEM((NTOT,), jnp.int32),
            pltpu.VMEM((NTOT,), jnp.int32),
            pltpu.VMEM((NTOT,), jnp.int32),
            pltpu.VMEM((NTOT,), jnp.int32),
            pltpu.VMEM((NBINS_MAX,), jnp.int32),
            pltpu.VMEM((2, NTOT), jnp.float32),
            pltpu.VMEM((2, NTOT), jnp.float32),
            pltpu.SemaphoreType.DMA,
            pltpu.SemaphoreType.DMA,
            pltpu.SemaphoreType.DMA,
            pltpu.SemaphoreType.DMA,
        ],
        compiler_params=pltpu.CompilerParams(needs_layout_passes=False),
    )
    def sort_gather(d_f_hbm, sel_hbm, keyf, key_a, key_b, idx_a, idx_b,
                    hist, data_v, out_v, sem_i0, sem_i1, sem_o0, sem_o1):
        cid = lax.axis_index("c")
        sid = lax.axis_index("s")
        wid = sid * nc + cid
        i16 = lax.iota(jnp.int32, 16)
        sems_i = (sem_i0, sem_i1)
        sems_o = (sem_o0, sem_o1)

        def do_row(r, carry_unused):
            b = wid * rows_per_w + r
            row0 = b * C
            pltpu.sync_copy(d_f_hbm.at[row0], keyf)

            def prep(i, _):
                sl = pl.ds(i * 16, 16)
                bits = plsc.bitcast(keyf[sl], jnp.int32)
                m = jnp.bitwise_and(bits, jnp.int32(0x7FFFFFFF))
                key_a[sl] = jnp.bitwise_xor(m, jnp.int32(-1))
                idx_a[sl] = i16 + i * 16
                return 0

            lax.fori_loop(0, NVEC, prep, 0)

            bufs = ((key_a, idx_a, key_b, idx_b),
                    (key_b, idx_b, key_a, idx_a),
                    (key_a, idx_a, key_b, idx_b))
            for p in range(3):
                shift, nbits = PASS_SHIFTS[p], PASS_BITS[p]
                nb = 1 << nbits
                src_k, src_i, dst_k, dst_i = bufs[p]

                def zero(j, _):
                    hist[pl.ds(j * 16, 16)] = jnp.zeros((16,), jnp.int32)
                    return 0

                lax.fori_loop(0, nb // 16, zero, 0)

                def histo(i, _):
                    sl = pl.ds(i * 16, 16)
                    d = _digit(src_k[sl], shift, nbits)
                    cnt, last = plsc.scan_count(d)
                    g = plsc.load_gather(hist, [d])
                    plsc.store_scatter(hist, [d], g + cnt, mask=last)
                    return 0

                lax.fori_loop(0, NVEC, histo, 0)

                def scan(j, carry):
                    sl = pl.ds(j * 16, 16)
                    v = hist[sl]
                    s = plsc.cumsum(v)
                    hist[sl] = carry + s - v
                    return carry + jnp.sum(v)

                lax.fori_loop(0, nb // 16, scan, jnp.int32(0))

                def permute(i, _):
                    sl = pl.ds(i * 16, 16)
                    k = src_k[sl]
                    iv = src_i[sl]
                    d = _digit(k, shift, nbits)
                    cnt, last = plsc.scan_count(d)
                    base = plsc.load_gather(hist, [d])
                    pos = base + cnt - 1
                    plsc.store_scatter(dst_k, [pos], k)
                    plsc.store_scatter(dst_i, [pos], iv)
                    plsc.store_scatter(hist, [d], base + cnt, mask=last)
                    return 0

                lax.fori_loop(0, NVEC, permute, 0)

            cp_in = [None, None]
            cp_out = [None, None]
            cp_in[0] = pltpu.async_copy(d_f_hbm.at[row0], data_v.at[0],
                                        sems_i[0])
            for c in range(C):
                buf = c % 2
                cp_in[buf].wait()
                if c + 1 < C:
                    nbuf = (c + 1) % 2
                    cp_in[nbuf] = pltpu.async_copy(
                        d_f_hbm.at[row0 + c + 1], data_v.at[nbuf],
                        sems_i[nbuf])
                if c >= 2:
                    cp_out[buf].wait()
                src = data_v.at[buf]
                dst = out_v.at[buf]

                def gath(i, _):
                    sl = pl.ds(i * 16, 16)
                    ids = idx_b[sl]
                    dst[sl] = plsc.load_gather(src, [ids])
                    return 0

                lax.fori_loop(0, NVEC, gath, 0)
                cp_out[buf] = pltpu.async_copy(dst, sel_hbm.at[row0 + c],
                                               sems_o[buf])
            cp_out[0].wait()
            cp_out[1].wait()
            return 0

        lax.fori_loop(0, rows_per_w, do_row, 0)

    return sort_gather(d_f2)


def _tc_mlp(sel, W1T, b1, W2, b2, W3T, b3, W4T, b4):
    BT = 64
    grid = (B // BT,)

    def body(sel_ref, W1T_ref, b1_ref, W2_ref, b2_ref, W3T_ref, b3_ref,
             W4T_ref, b4_ref, out_ref):
        s_cols = []
        for c in range(C):
            x = sel_ref[:, c, :]
            h = jnp.dot(x, W1T_ref[c],
                        preferred_element_type=jnp.float32)
            h = jnp.maximum(h + b1_ref[c][None, :], 0.0)
            s_c = jnp.sum(h * W2_ref[c][None, :], axis=1, keepdims=True)
            s_cols.append(s_c)
        s = jnp.concatenate(s_cols, axis=1) + b2_ref[...]
        z = jnp.dot(s, W3T_ref[...], preferred_element_type=jnp.float32)
        z = jnp.maximum(z + b3_ref[...], 0.0)
        out = jnp.dot(z, W4T_ref[...],
                      preferred_element_type=jnp.float32) + b4_ref[...]
        out_ref[...] = out

    return pl.pallas_call(
        body,
        grid=grid,
        in_specs=[
            pl.BlockSpec((BT, C, NTOT), lambda i: (i, 0, 0)),
            pl.BlockSpec((C, NTOT, 10), lambda i: (0, 0, 0)),
            pl.BlockSpec((C, 10), lambda i: (0, 0)),
            pl.BlockSpec((C, 10), lambda i: (0, 0)),
            pl.BlockSpec((1, C), lambda i: (0, 0)),
            pl.BlockSpec((C, 30), lambda i: (0, 0)),
            pl.BlockSpec((1, 30), lambda i: (0, 0)),
            pl.BlockSpec((30, 128), lambda i: (0, 0)),
            pl.BlockSpec((1, 128), lambda i: (0, 0)),
        ],
        out_specs=pl.BlockSpec((BT, 128), lambda i: (i, 0)),
        out_shape=jax.ShapeDtypeStruct((B, 128), jnp.float32),
    )(sel, W1T, b1, W2, b2, W3T, b3, W4T, b4)


def kernel(d_f, d_t, W1, b1, W2, b2, W3, b3, W4, b4):
    d_f2 = d_f.reshape(B * C, NTOT)
    sel2 = _sc_sort_gather(d_f2)
    sel = sel2.reshape(B, C, NTOT)
    compressed = _tc_mlp(
        sel,
        W1.transpose(0, 2, 1),
        b1,
        W2[:, 0, :],
        b2[:, 0][None, :],
        W3.transpose(1, 0),
        b3[None, :],
        W4.transpose(1, 0),
        b4[None, :],
    )
    return (compressed, d_t)


# speedup vs baseline: 630.3683x; 630.3683x over previous
import functools

import jax
import jax.numpy as jnp
from jax import lax
from jax.experimental import pallas as pl
from jax.experimental.pallas import tpu as pltpu
from jax.experimental.pallas import tpu_sc as plsc

B, C, NTOT = 1024, 16, 4096
NVEC = NTOT // 16
PASS_SHIFTS = (0, 11, 22)
PASS_BITS = (11, 11, 10)
NBINS_MAX = 1 << 11


def _digit(u, shift, nbits):
    mask = (1 << nbits) - 1
    return jnp.bitwise_and(lax.shift_right_logical(u, jnp.int32(shift)),
                           jnp.int32(mask))


def _sc_sort_gather(d_f2):
    info = plsc.get_sparse_core_info()
    nc, ns = info.num_cores, info.num_subcores
    nw = nc * ns
    assert B % nw == 0
    rows_per_w = B // nw

    mesh = plsc.VectorSubcoreMesh(core_axis_name="c", subcore_axis_name="s")

    @functools.partial(
        pl.kernel,
        mesh=mesh,
        out_type=jax.ShapeDtypeStruct((B * C, NTOT), jnp.float32),
        scratch_types=[
            pltpu.VMEM((NTOT,), jnp.float32),
            pltpu.VMEM((NTOT,), jnp.int32),
            pltpu.VMEM((NTOT,), jnp.int32),
            pltpu.VMEM((NTOT,), jnp.int32),
            pltpu.VMEM((NTOT,), jnp.int32),
            pltpu.VMEM((NBINS_MAX,), jnp.int32),
            pltpu.VMEM((NTOT,), jnp.float32),
            pltpu.VMEM((NTOT,), jnp.float32),
            pltpu.VMEM((NTOT,), jnp.float32),
            pltpu.VMEM((NTOT,), jnp.float32),
            pltpu.SemaphoreType.DMA,
            pltpu.SemaphoreType.DMA,
            pltpu.SemaphoreType.DMA,
            pltpu.SemaphoreType.DMA,
        ],
        compiler_params=pltpu.CompilerParams(needs_layout_passes=False),
    )
    def sort_gather(d_f_hbm, sel_hbm, keyf, key_a, key_b, idx_a, idx_b,
                    hist, data_v0, data_v1, out_v0, out_v1,
                    sem_i0, sem_i1, sem_o0, sem_o1):
        cid = lax.axis_index("c")
        sid = lax.axis_index("s")
        wid = sid * nc + cid
        i16 = lax.iota(jnp.int32, 16)
        sems_i = (sem_i0, sem_i1)
        sems_o = (sem_o0, sem_o1)
        data_v = (data_v0, data_v1)
        out_v = (out_v0, out_v1)

        def do_row(r, carry_unused):
            b = wid * rows_per_w + r
            row0 = b * C
            pltpu.sync_copy(d_f_hbm.at[row0], keyf)

            def prep(i, _):
                sl = pl.ds(i * 16, 16)
                bits = plsc.bitcast(keyf[sl], jnp.int32)
                m = jnp.bitwise_and(bits, jnp.int32(0x7FFFFFFF))
                key_a[sl] = jnp.bitwise_xor(m, jnp.int32(-1))
                idx_a[sl] = i16 + i * 16
                return 0

            lax.fori_loop(0, NVEC, prep, 0)

            bufs = ((key_a, idx_a, key_b, idx_b),
                    (key_b, idx_b, key_a, idx_a),
                    (key_a, idx_a, key_b, idx_b))
            for p in range(3):
                shift, nbits = PASS_SHIFTS[p], PASS_BITS[p]
                nb = 1 << nbits
                src_k, src_i, dst_k, dst_i = bufs[p]

                def zero(j, _):
                    hist[pl.ds(j * 16, 16)] = jnp.zeros((16,), jnp.int32)
                    return 0

                lax.fori_loop(0, nb // 16, zero, 0)

                def histo(i, _):
                    sl = pl.ds(i * 16, 16)
                    d = _digit(src_k[sl], shift, nbits)
                    cnt, last = plsc.scan_count(d)
                    g = plsc.load_gather(hist, [d])
                    plsc.store_scatter(hist, [d], g + cnt, mask=last)
                    return 0

                lax.fori_loop(0, NVEC, histo, 0)

                def scan(j, carry):
                    sl = pl.ds(j * 16, 16)
                    v = hist[sl]
                    s = plsc.cumsum(v)
                    hist[sl] = carry + s - v
                    return carry + jnp.sum(v)

                lax.fori_loop(0, nb // 16, scan, jnp.int32(0))

                def permute(i, _):
                    sl = pl.ds(i * 16, 16)
                    k = src_k[sl]
                    iv = src_i[sl]
                    d = _digit(k, shift, nbits)
                    cnt, last = plsc.scan_count(d)
                    base = plsc.load_gather(hist, [d])
                    pos = base + cnt - 1
                    plsc.store_scatter(dst_k, [pos], k)
                    plsc.store_scatter(dst_i, [pos], iv)
                    plsc.store_scatter(hist, [d], base + cnt, mask=last)
                    return 0

                lax.fori_loop(0, NVEC, permute, 0)

            cp_in = [None, None]
            cp_out = [None, None]
            cp_in[0] = pltpu.async_copy(d_f_hbm.at[row0], data_v[0],
                                        sems_i[0])
            for c in range(C):
                buf = c % 2
                cp_in[buf].wait()
                if c + 1 < C:
                    nbuf = (c + 1) % 2
                    cp_in[nbuf] = pltpu.async_copy(
                        d_f_hbm.at[row0 + c + 1], data_v[nbuf],
                        sems_i[nbuf])
                if c >= 2:
                    cp_out[buf].wait()
                src = data_v[buf]
                dst = out_v[buf]

                def gath(i, _):
                    sl = pl.ds(i * 16, 16)
                    ids = idx_b[sl]
                    dst[sl] = plsc.load_gather(src, [ids])
                    return 0

                lax.fori_loop(0, NVEC, gath, 0)
                cp_out[buf] = pltpu.async_copy(dst, sel_hbm.at[row0 + c],
                                               sems_o[buf])
            cp_out[0].wait()
            cp_out[1].wait()
            return 0

        lax.fori_loop(0, rows_per_w, do_row, 0)

    return sort_gather(d_f2)


def _tc_mlp(sel, W1, b1, W2, b2, W3T, b3, W4T, b4):
    BT = 64
    grid = (B // BT,)

    def body(sel_ref, W1_ref, b1_ref, W2_ref, b2_ref, W3T_ref, b3_ref,
             W4T_ref, b4_ref, out_ref):
        s_cols = []
        for c in range(C):
            x = sel_ref[:, c, :]
            h = lax.dot_general(x, W1_ref[c],
                                (((1,), (1,)), ((), ())),
                                preferred_element_type=jnp.float32)
            h = jnp.maximum(h + b1_ref[c][None, :], 0.0)
            s_c = jnp.sum(h * W2_ref[c][None, :], axis=1, keepdims=True)
            s_cols.append(s_c)
        s = jnp.concatenate(s_cols, axis=1) + b2_ref[...]
        z = jnp.dot(s, W3T_ref[...], preferred_element_type=jnp.float32)
        z = jnp.maximum(z + b3_ref[...], 0.0)
        out = jnp.dot(z, W4T_ref[...],
                      preferred_element_type=jnp.float32) + b4_ref[...]
        out_ref[...] = out

    return pl.pallas_call(
        body,
        grid=grid,
        in_specs=[
            pl.BlockSpec((BT, C, NTOT), lambda i: (i, 0, 0)),
            pl.BlockSpec((C, 10, NTOT), lambda i: (0, 0, 0)),
            pl.BlockSpec((C, 10), lambda i: (0, 0)),
            pl.BlockSpec((C, 10), lambda i: (0, 0)),
            pl.BlockSpec((1, C), lambda i: (0, 0)),
            pl.BlockSpec((C, 30), lambda i: (0, 0)),
            pl.BlockSpec((1, 30), lambda i: (0, 0)),
            pl.BlockSpec((30, 128), lambda i: (0, 0)),
            pl.BlockSpec((1, 128), lambda i: (0, 0)),
        ],
        out_specs=pl.BlockSpec((BT, 128), lambda i: (i, 0)),
        out_shape=jax.ShapeDtypeStruct((B, 128), jnp.float32),
    )(sel, W1, b1, W2, b2, W3T, b3, W4T, b4)


def kernel(d_f, d_t, W1, b1, W2, b2, W3, b3, W4, b4):
    d_f2 = d_f.reshape(B * C, NTOT)
    sel2 = _sc_sort_gather(d_f2)
    sel = sel2.reshape(B, C, NTOT)
    compressed = _tc_mlp(
        sel,
        W1,
        b1,
        W2[:, 0, :],
        b2[:, 0][None, :],
        W3.transpose(1, 0),
        b3[None, :],
        W4.transpose(1, 0),
        b4[None, :],
    )
    return (compressed, d_t)


# speedup vs baseline: 756.8973x; 1.2007x over previous
import functools

import jax
import jax.numpy as jnp
from jax import lax
from jax.experimental import pallas as pl
from jax.experimental.pallas import tpu as pltpu
from jax.experimental.pallas import tpu_sc as plsc

B, C, NTOT = 1024, 16, 4096
NVEC = NTOT // 16
PASS_SHIFTS = (0, 11, 22)
PASS_BITS = (11, 11, 10)
NBINS_MAX = 1 << 11


def _digit(u, shift, nbits):
    mask = (1 << nbits) - 1
    return jnp.bitwise_and(lax.shift_right_logical(u, jnp.int32(shift)),
                           jnp.int32(mask))


def _sc_sort_gather(d_f2):
    info = plsc.get_sparse_core_info()
    nc, ns = info.num_cores, info.num_subcores
    nw = nc * ns
    assert B % nw == 0
    rows_per_w = B // nw

    mesh = plsc.VectorSubcoreMesh(core_axis_name="c", subcore_axis_name="s")

    @functools.partial(
        pl.kernel,
        mesh=mesh,
        out_type=jax.ShapeDtypeStruct((B * C, NTOT), jnp.float32),
        scratch_types=[
            pltpu.VMEM((NTOT,), jnp.float32),
            pltpu.VMEM((NTOT,), jnp.int32),
            pltpu.VMEM((NTOT,), jnp.int32),
            pltpu.VMEM((NTOT,), jnp.int32),
            pltpu.VMEM((NTOT,), jnp.int32),
            pltpu.VMEM((NBINS_MAX,), jnp.int32),
            pltpu.VMEM((NBINS_MAX,), jnp.int32),
            pltpu.VMEM((NBINS_MAX,), jnp.int32),
            pltpu.VMEM((NBINS_MAX,), jnp.int32),
            pltpu.VMEM((NBINS_MAX,), jnp.int32),
            pltpu.VMEM((NBINS_MAX,), jnp.int32),
            pltpu.VMEM((NBINS_MAX,), jnp.int32),
            pltpu.VMEM((NBINS_MAX,), jnp.int32),
            pltpu.VMEM((NTOT,), jnp.float32),
            pltpu.VMEM((NTOT,), jnp.float32),
            pltpu.VMEM((NTOT,), jnp.float32),
            pltpu.VMEM((NTOT,), jnp.float32),
            pltpu.SemaphoreType.DMA,
            pltpu.SemaphoreType.DMA,
            pltpu.SemaphoreType.DMA,
            pltpu.SemaphoreType.DMA,
        ],
        compiler_params=pltpu.CompilerParams(needs_layout_passes=False),
    )
    def sort_gather(d_f_hbm, sel_hbm, keyf, key_a, key_b, idx_a, idx_b,
                    h0, h1, h2, h3, o0, o1, o2, o3,
                    data_v0, data_v1, out_v0, out_v1,
                    sem_i0, sem_i1, sem_o0, sem_o1):
        cid = lax.axis_index("c")
        sid = lax.axis_index("s")
        wid = sid * nc + cid
        i16 = lax.iota(jnp.int32, 16)
        sems_i = (sem_i0, sem_i1)
        sems_o = (sem_o0, sem_o1)
        data_v = (data_v0, data_v1)
        out_v = (out_v0, out_v1)
        hists = (h0, h1, h2, h3)
        offs = (o0, o1, o2, o3)
        NSLOT = 4
        CHUNK = NVEC // NSLOT

        def do_row(r, carry_unused):
            b = wid * rows_per_w + r
            row0 = b * C
            pltpu.sync_copy(d_f_hbm.at[row0], keyf)

            bufs = ((key_a, idx_a, key_b, idx_b),
                    (key_b, idx_b, key_a, idx_a),
                    (key_a, idx_a, key_b, idx_b))
            for p in range(3):
                shift, nbits = PASS_SHIFTS[p], PASS_BITS[p]
                nb = 1 << nbits
                src_k, src_i, dst_k, dst_i = bufs[p]

                def zero(j, _):
                    sl = pl.ds(j * 16, 16)
                    z = jnp.zeros((16,), jnp.int32)
                    for h in hists:
                        h[sl] = z
                    return 0

                lax.fori_loop(0, nb // 16, zero, 0)

                if p == 0:
                    def histo(it, _):
                        for j in range(NSLOT):
                            i = j * CHUNK + it
                            sl = pl.ds(i * 16, 16)
                            bits = plsc.bitcast(keyf[sl], jnp.int32)
                            m = jnp.bitwise_and(bits, jnp.int32(0x7FFFFFFF))
                            u = jnp.bitwise_xor(m, jnp.int32(-1))
                            key_a[sl] = u
                            idx_a[sl] = i16 + i * 16
                            d = _digit(u, shift, nbits)
                            cnt, last = plsc.scan_count(d)
                            g = plsc.load_gather(hists[j], [d])
                            plsc.store_scatter(hists[j], [d], g + cnt,
                                               mask=last)
                        return 0
                else:
                    def histo(it, _):
                        for j in range(NSLOT):
                            i = j * CHUNK + it
                            sl = pl.ds(i * 16, 16)
                            d = _digit(src_k[sl], shift, nbits)
                            cnt, last = plsc.scan_count(d)
                            g = plsc.load_gather(hists[j], [d])
                            plsc.store_scatter(hists[j], [d], g + cnt,
                                               mask=last)
                        return 0

                lax.fori_loop(0, CHUNK, histo, 0)

                def scan(j, carry):
                    sl = pl.ds(j * 16, 16)
                    v0 = h0[sl]
                    v1 = h1[sl]
                    v2 = h2[sl]
                    v3 = h3[sl]
                    t01 = v0 + v1
                    tot = t01 + v2 + v3
                    s = plsc.cumsum(tot)
                    excl = carry + s - tot
                    o0[sl] = excl
                    o1[sl] = excl + v0
                    o2[sl] = excl + t01
                    o3[sl] = excl + t01 + v2
                    return carry + jnp.sum(tot)

                lax.fori_loop(0, nb // 16, scan, jnp.int32(0))

                last_pass = p == 2

                def permute(it, _):
                    for j in range(NSLOT):
                        i = j * CHUNK + it
                        sl = pl.ds(i * 16, 16)
                        k = src_k[sl]
                        iv = src_i[sl]
                        d = _digit(k, shift, nbits)
                        cnt, last = plsc.scan_count(d)
                        base = plsc.load_gather(offs[j], [d])
                        pos = base + cnt - 1
                        if not last_pass:
                            plsc.store_scatter(dst_k, [pos], k)
                        plsc.store_scatter(dst_i, [pos], iv)
                        plsc.store_scatter(offs[j], [d], base + cnt,
                                           mask=last)
                    return 0

                lax.fori_loop(0, CHUNK, permute, 0)

            cp_in = [None, None]
            cp_out = [None, None]
            cp_in[0] = pltpu.async_copy(d_f_hbm.at[row0], data_v[0],
                                        sems_i[0])
            for c in range(C):
                buf = c % 2
                cp_in[buf].wait()
                if c + 1 < C:
                    nbuf = (c + 1) % 2
                    cp_in[nbuf] = pltpu.async_copy(
                        d_f_hbm.at[row0 + c + 1], data_v[nbuf],
                        sems_i[nbuf])
                if c >= 2:
                    cp_out[buf].wait()
                src = data_v[buf]
                dst = out_v[buf]

                def gath(it, _):
                    for u in range(4):
                        sl = pl.ds((it * 4 + u) * 16, 16)
                        ids = idx_b[sl]
                        dst[sl] = plsc.load_gather(src, [ids])
                    return 0

                lax.fori_loop(0, NVEC // 4, gath, 0)
                cp_out[buf] = pltpu.async_copy(dst, sel_hbm.at[row0 + c],
                                               sems_o[buf])
            cp_out[0].wait()
            cp_out[1].wait()
            return 0

        lax.fori_loop(0, rows_per_w, do_row, 0)

    return sort_gather(d_f2)


def _tc_mlp(sel, W1, b1, W2, b2, W3T, b3, W4T, b4):
    BT = 64
    grid = (B // BT,)

    def body(sel_ref, W1_ref, b1_ref, W2_ref, b2_ref, W3T_ref, b3_ref,
             W4T_ref, b4_ref, out_ref):
        s_cols = []
        for c in range(C):
            x = sel_ref[:, c, :]
            h = lax.dot_general(x, W1_ref[c],
                                (((1,), (1,)), ((), ())),
                                preferred_element_type=jnp.float32)
            h = jnp.maximum(h + b1_ref[c][None, :], 0.0)
            s_c = jnp.sum(h * W2_ref[c][None, :], axis=1, keepdims=True)
            s_cols.append(s_c)
        s = jnp.concatenate(s_cols, axis=1) + b2_ref[...]
        z = jnp.dot(s, W3T_ref[...], preferred_element_type=jnp.float32)
        z = jnp.maximum(z + b3_ref[...], 0.0)
        out = jnp.dot(z, W4T_ref[...],
                      preferred_element_type=jnp.float32) + b4_ref[...]
        out_ref[...] = out

    return pl.pallas_call(
        body,
        grid=grid,
        in_specs=[
            pl.BlockSpec((BT, C, NTOT), lambda i: (i, 0, 0)),
            pl.BlockSpec((C, 10, NTOT), lambda i: (0, 0, 0)),
            pl.BlockSpec((C, 10), lambda i: (0, 0)),
            pl.BlockSpec((C, 10), lambda i: (0, 0)),
            pl.BlockSpec((1, C), lambda i: (0, 0)),
            pl.BlockSpec((C, 30), lambda i: (0, 0)),
            pl.BlockSpec((1, 30), lambda i: (0, 0)),
            pl.BlockSpec((30, 128), lambda i: (0, 0)),
            pl.BlockSpec((1, 128), lambda i: (0, 0)),
        ],
        out_specs=pl.BlockSpec((BT, 128), lambda i: (i, 0)),
        out_shape=jax.ShapeDtypeStruct((B, 128), jnp.float32),
    )(sel, W1, b1, W2, b2, W3T, b3, W4T, b4)


def kernel(d_f, d_t, W1, b1, W2, b2, W3, b3, W4, b4):
    d_f2 = d_f.reshape(B * C, NTOT)
    sel2 = _sc_sort_gather(d_f2)
    sel = sel2.reshape(B, C, NTOT)
    compressed = _tc_mlp(
        sel,
        W1,
        b1,
        W2[:, 0, :],
        b2[:, 0][None, :],
        W3.transpose(1, 0),
        b3[None, :],
        W4.transpose(1, 0),
        b4[None, :],
    )
    return (compressed, d_t)
